# baseline (speedup 1.0000x reference)
import jax

jax.config.update("jax_enable_x64", True)

import jax.numpy as jnp
from jax import lax
from jax.experimental import pallas as pl
from jax.experimental.pallas import tpu as pltpu
from jax.experimental.pallas import tpu_sc as plsc

N = 16384
PATHS = 16
NUM_WORKERS = 32
TOK_PER_W = N // NUM_WORKERS


def _top2_body(scoreT_hbm, out_hbm, in_v, out_v):
    c = lax.axis_index("c")
    s = lax.axis_index("s")
    wid = s * 2 + c
    base = wid * TOK_PER_W
    pltpu.sync_copy(scoreT_hbm.at[:, pl.ds(base, TOK_PER_W)], in_v)

    one = jnp.full((16,), 1, jnp.uint32)
    zero = jnp.full((16,), 0, jnp.uint32)
    two = jnp.full((16,), 2, jnp.uint32)
    neginf = jnp.full((16,), -jnp.inf, jnp.float32)

    def one_block(col0):
        cols = [in_v[p, pl.ds(col0, 16)] for p in range(PATHS)]
        m1 = cols[0]
        m2 = neginf
        for p in range(1, PATHS):
            t = jnp.minimum(m1, cols[p])
            m2 = jnp.maximum(m2, t)
            m1 = jnp.maximum(m1, cols[p])
        need = jnp.where(m1 > m2, one, two)
        cnt = zero
        for p in range(PATHS):
            eqi = jnp.where(cols[p] == m2, one, zero)
            gti = jnp.where(cols[p] > m2, one, zero)
            oki = jnp.where(cnt < need, one, zero)
            out_v[p, pl.ds(col0, 16)] = gti | (eqi & oki)
            cnt = cnt + eqi

    def blk(b, carry):
        one_block(b * 64)
        one_block(b * 64 + 16)
        one_block(b * 64 + 32)
        one_block(b * 64 + 48)
        return carry

    lax.fori_loop(jnp.int32(0), jnp.int32(TOK_PER_W // 64), blk, 0)
    pltpu.sync_copy(out_v, out_hbm.at[:, pl.ds(base, TOK_PER_W)])


def kernel(score):
    mesh = plsc.VectorSubcoreMesh(core_axis_name="c", subcore_axis_name="s")
    k = pl.kernel(
        _top2_body,
        mesh=mesh,
        out_type=jax.ShapeDtypeStruct((PATHS, N), jnp.uint32),
        scratch_types=[
            pltpu.VMEM((PATHS, TOK_PER_W), jnp.float32),
            pltpu.VMEM((PATHS, TOK_PER_W), jnp.uint32),
        ],
    )
    return k(score.T).T.astype(jnp.int64)


# speedup vs baseline: 1.2580x; 1.2580x over previous
import jax

jax.config.update("jax_enable_x64", True)

import jax.numpy as jnp
from jax import lax
from jax.experimental import pallas as pl
from jax.experimental.pallas import tpu as pltpu
from jax.experimental.pallas import tpu_sc as plsc

N = 16384
PATHS = 16
NUM_WORKERS = 32
TOK_PER_W = N // NUM_WORKERS


def _top2_body(scoreT_hbm, out_hbm, in_v, out_v):
    c = lax.axis_index("c")
    s = lax.axis_index("s")
    wid = s * 2 + c
    base = wid * TOK_PER_W
    pltpu.sync_copy(scoreT_hbm.at[:, pl.ds(base, TOK_PER_W)], in_v)

    one = jnp.full((16,), 1, jnp.int32)
    zero = jnp.full((16,), 0, jnp.int32)
    two = jnp.full((16,), 2, jnp.int32)
    neginf = jnp.full((16,), -jnp.inf, jnp.float32)

    def one_block(col0):
        cols = [in_v[p, pl.ds(col0, 16)] for p in range(PATHS)]
        m1 = cols[0]
        m2 = neginf
        for p in range(1, PATHS):
            t = jnp.minimum(m1, cols[p])
            m2 = jnp.maximum(m2, t)
            m1 = jnp.maximum(m1, cols[p])
        need = jnp.where(m1 > m2, one, two)
        cnt = zero
        for p in range(PATHS):
            eqi = jnp.where(cols[p] == m2, one, zero)
            gti = jnp.where(cols[p] > m2, one, zero)
            oki = jnp.where(cnt < need, one, zero)
            out_v[p, pl.ds(col0, 16)] = gti | (eqi & oki)
            cnt = cnt + eqi

    def blk(b, carry):
        one_block(b * 64)
        one_block(b * 64 + 16)
        one_block(b * 64 + 32)
        one_block(b * 64 + 48)
        return carry

    lax.fori_loop(jnp.int32(0), jnp.int32(TOK_PER_W // 64), blk, 0)
    pltpu.sync_copy(out_v, out_hbm.at[:, pl.ds(base, TOK_PER_W)])


def kernel(score):
    mesh = plsc.VectorSubcoreMesh(core_axis_name="c", subcore_axis_name="s")
    k = pl.kernel(
        _top2_body,
        mesh=mesh,
        out_type=jax.ShapeDtypeStruct((PATHS, N), jnp.int32),
        scratch_types=[
            pltpu.VMEM((PATHS, TOK_PER_W), jnp.float32),
            pltpu.VMEM((PATHS, TOK_PER_W), jnp.int32),
        ],
    )
    return k(score.T).T.astype(jnp.int64)


# speedup vs baseline: 1.2816x; 1.0187x over previous
import jax

jax.config.update("jax_enable_x64", True)

import jax.numpy as jnp
from jax import lax
from jax.experimental import pallas as pl
from jax.experimental.pallas import tpu as pltpu
from jax.experimental.pallas import tpu_sc as plsc

N = 16384
PATHS = 16
NUM_WORKERS = 32
TOK_PER_W = N // NUM_WORKERS


def _top2_body(scoreT_hbm, out_hbm, in_v, out_v):
    c = lax.axis_index("c")
    s = lax.axis_index("s")
    wid = s * 2 + c
    base = wid * TOK_PER_W
    pltpu.sync_copy(scoreT_hbm.at[:, pl.ds(base, TOK_PER_W)], in_v)

    one = jnp.full((16,), 1, jnp.int32)
    zero = jnp.full((16,), 0, jnp.int32)
    two = jnp.full((16,), 2, jnp.int32)
    neginf = jnp.full((16,), -jnp.inf, jnp.float32)

    def one_block(col0):
        cols = [in_v[p, pl.ds(col0, 16)] for p in range(PATHS)]
        m1 = cols[0]
        m2 = neginf
        for p in range(1, PATHS):
            t = jnp.minimum(m1, cols[p])
            m2 = jnp.maximum(m2, t)
            m1 = jnp.maximum(m1, cols[p])
        need = jnp.where(m1 > m2, one, two)
        cnt = zero
        for p in range(PATHS):
            eqi = jnp.where(cols[p] == m2, one, zero)
            gti = jnp.where(cols[p] > m2, one, zero)
            oki = jnp.where(cnt < need, one, zero)
            out_v[p, pl.ds(col0, 16)] = gti | (eqi & oki)
            cnt = cnt + eqi

    def blk(b, carry):
        one_block(b * 32)
        one_block(b * 32 + 16)
        return carry

    lax.fori_loop(jnp.int32(0), jnp.int32(TOK_PER_W // 32), blk, 0)
    pltpu.sync_copy(out_v, out_hbm.at[:, pl.ds(base, TOK_PER_W)])


def kernel(score):
    mesh = plsc.VectorSubcoreMesh(core_axis_name="c", subcore_axis_name="s")
    k = pl.kernel(
        _top2_body,
        mesh=mesh,
        out_type=jax.ShapeDtypeStruct((PATHS, N), jnp.int32),
        scratch_types=[
            pltpu.VMEM((PATHS, TOK_PER_W), jnp.float32),
            pltpu.VMEM((PATHS, TOK_PER_W), jnp.int32),
        ],
    )
    return k(score.T).T.astype(jnp.int64)
